# baseline (speedup 1.0000x reference)
import functools

import jax
import jax.numpy as jnp
from jax import lax
from jax.experimental import pallas as pl
from jax.experimental.pallas import tpu as pltpu
from jax.experimental.pallas import tpu_sc as plsc

_Q, _K, _D = 1024, 16384, 256
_T, _H = 8, 16
_KOUT = 100
_OPAD = 112
_CAP = 256
_ROW = _CAP + 16
_NSUB = 32

_NEG = -1e9
_NINF = -3.0e38


def _codes_body(x_ref, wf_ref, bf_ref, p2_ref, code_ref):
    x = x_ref[...]
    proj = jnp.dot(x, wf_ref[...], preferred_element_type=jnp.float32)
    proj = proj + bf_ref[...]
    bits = (proj > 0).astype(jnp.float32)
    codef = jnp.dot(bits, p2_ref[...], preferred_element_type=jnp.float32)
    code_ref[...] = codef.astype(jnp.int32)


def _sims_body(q_ref, k_ref, o_ref):
    q = q_ref[...]
    for j in range(8):
        kb = k_ref[pl.ds(j * 128, 128), :]
        o_ref[:, j, :] = lax.dot_general(
            q, kb, (((1,), (1,)), ((), ())),
            preferred_element_type=jnp.float32)


def _iota16():
    return lax.iota(jnp.int32, 16)


def _scount(mask):
    return plsc.all_reduce_population_count(mask)[0]


def _sload(ref, i):
    return ref[pl.ds(i, 16)][0]


def _sstore(ref, i, val):
    old = ref[pl.ds(i, 16)]
    vv = jnp.full((16,), val)
    ref[pl.ds(i, 16)] = jnp.where(lax.iota(jnp.int32, 16) == 0, vv, old)


def _compact(m, payload):
    _, sv, _ = plsc.sort_key_val(lax.iota(jnp.int32, 16), payload, mask=m)
    return sv


def _match_body(cq_ref, ck_ref, pp_ref,
                qtab, nxt, qc, kc, hits, pp, sem):
    wid = lax.axis_index("s") * 2 + lax.axis_index("c")
    t = wid // 4
    quarter = wid % 4
    i16 = _iota16()
    neg16 = jnp.full((16,), -1, jnp.int32)

    pltpu.async_copy(cq_ref.at[pl.ds(t * _Q, _Q)],
                     qc.at[pl.ds(0, _Q)], sem).wait()
    pltpu.async_copy(ck_ref.at[pl.ds(t * _K + quarter * 4096, 4096)],
                     kc.at[pl.ds(0, 4096)], sem).wait()

    def zq(i, _):
        qtab[pl.ds(i * 16, 16)] = neg16
        return 0
    lax.fori_loop(0, 4096, zq, 0)

    def zpair(i, _):
        pp[pl.ds(i * 16, 16)] = neg16
        return 0
    lax.fori_loop(0, _ROW // 16, zpair, 0)

    def setbit(i, _):
        c = _sload(qc, i)
        _sstore(nxt, i, _sload(qtab, c))
        _sstore(qtab, c, i)
        return 0
    lax.fori_loop(0, _Q, setbit, 0)

    def scan(i, nh):
        k16 = kc[pl.ds(i * 16, 16)]
        wv = plsc.load_gather(qtab, [k16])
        hit = wv >= 0
        cnt = _scount(hit)

        @pl.when(cnt > 0)
        def _():
            hits[pl.ds(nh, 16)] = _compact(hit, i16 + i * 16)
        return nh + cnt
    nh = lax.fori_loop(0, 256, scan, 0)

    def per_hit(j, np_):
        hk = _sload(hits, j)
        ck = _sload(kc, hk)
        kg = hk + quarter * 4096

        def chain_cond(st):
            q, _n = st
            return q >= 0

        def chain_body(st):
            q, n = st
            _sstore(pp, jnp.minimum(n, _CAP), q * _K + kg)
            return _sload(nxt, q), n + 1
        _, np2 = lax.while_loop(chain_cond, chain_body,
                                (_sload(qtab, ck), np_))
        return np2
    lax.fori_loop(0, nh, per_hit, 0)

    pltpu.async_copy(pp, pp_ref.at[pl.ds(wid * _ROW, _ROW)], sem).wait()


def _select_body(pp_ref, sims_ref, ov_ref, oi_ref,
                 ap, myp, vals, rowidx, rows, buf2d, vbuf, cnt, shv, shp,
                 fl, fb, ovb, oib, sem):
    wid = lax.axis_index("s") * 2 + lax.axis_index("c")
    lo = wid * 32
    i16 = _iota16()
    zero16 = jnp.zeros((16,), jnp.int32)
    neg16 = jnp.full((16,), -1, jnp.int32)
    ninf16 = jnp.full((16,), _NINF, jnp.float32)
    negv16 = jnp.full((16,), _NEG, jnp.float32)

    with jax.named_scope("b_dma"):
        pltpu.async_copy(pp_ref, ap, sem).wait()

    def zinit(i, _):
        myp[pl.ds(i * 16, 16)] = neg16
        rowidx[pl.ds(i * 16, 16)] = i16 + i * 16
        return 0
    lax.fori_loop(0, _ROW // 16, zinit, 0)

    plo = lo * _K
    phi = (lo + 32) * _K

    def filt(c, np_):
        ps = ap[pl.ds(c * 16, 16)]
        m = (ps >= plo) & (ps < phi)
        myp[pl.ds(jnp.minimum(np_, _CAP), 16)] = _compact(m, ps)
        return np_ + _scount(m)
    with jax.named_scope("b_filt"):
        np_ = lax.fori_loop(0, _NSUB * _ROW // 16, filt, 0)
    tail = jnp.minimum(np_, _CAP)
    myp[pl.ds(tail, 16)] = neg16
    nch = (tail + 15) // 16

    def rowix(c, _):
        ps = myp[pl.ds(c * 16, 16)]
        r = jnp.where(ps >= 0, (ps >> 14) * 128 + ((ps & 16383) >> 7),
                      i16 + c * 16)
        rowidx[pl.ds(c * 16, 16)] = r
        return 0

    with jax.named_scope("b_rowix"):
        lax.fori_loop(0, nch, rowix, 0)
    for j in range(_CAP // 16):
        @pl.when(j * 16 < tail)
        def _issue(j=j):
            pltpu.async_copy(
                sims_ref.at[rowidx.at[pl.ds(j * 16, 16)]],
                rows.at[pl.ds(j * 16, 16)], sem)
    with jax.named_scope("b_gather"):
        for j in range(_CAP // 16):
            @pl.when(j * 16 < tail)
            def _drain(j=j):
                pltpu.make_async_copy(
                    sims_ref.at[rowidx.at[pl.ds(j * 16, 16)]],
                    rows.at[pl.ds(j * 16, 16)], sem).wait()

    def getval(c, _):
        pi = i16 + c * 16
        ps = myp[pl.ds(c * 16, 16)]
        v = plsc.load_gather(rows, [pi, ps & 127])
        vals[pl.ds(c * 16, 16)] = jnp.where(ps >= 0, v, ninf16)
        return 0
    lax.fori_loop(0, nch, getval, 0)

    def zcnt(i, _):
        cnt[pl.ds(i * 16, 16)] = zero16
        return 0
    lax.fori_loop(0, 3, zcnt, 0)

    def bpass(i, _):
        p = _sload(myp, i)
        lq = (p >> 14) - lo
        cv = _sload(cnt, lq)
        slot = lq * 16 + jnp.minimum(cv, 15)
        _sstore(buf2d, slot, p)
        _sstore(vbuf, slot, _sload(vals, i))
        _sstore(cnt, lq, cv + 1)
        return 0
    with jax.named_scope("b_bpass"):
        lax.fori_loop(0, jnp.minimum(np_, _CAP), bpass, 0)

    def per_query(lq, _):
        gq = lo + lq
        base = lq * _OPAD
        qlo = gq * _K
        qhi = qlo + _K

        c0 = _sload(cnt, lq)
        kv = buf2d[pl.ds(lq * 16, 16)]
        vv = vbuf[pl.ds(lq * 16, 16)]
        valid = i16 < c0
        sk, sp, _om = plsc.sort_key_val(vv, kv, mask=valid, descending=True)
        shv[pl.ds(0, 16)] = jnp.full((16,), 3.0e38, jnp.float32)
        shv[pl.ds(1, 16)] = sk
        shp[pl.ds(0, 16)] = neg16
        shp[pl.ds(1, 16)] = sp
        prevv = shv[pl.ds(0, 16)]
        prevp = shp[pl.ds(0, 16)]
        tie = valid & (sk == prevv) & (sp != prevp)
        dup = valid & (sp == prevp)
        keep = valid & (~dup)
        kk16 = sp & 16383
        anytie = _scount(tie) > 0
        anysmall = _scount(keep & (kk16 < 256)) > 0
        fastok = (c0 <= 16) & (~anytie) & (~anysmall)

        @pl.when(fastok)
        def _fast_q():
            e = _scount(keep)
            oib[pl.ds(base, 16)] = _compact(keep, kk16)
            ovb[pl.ds(base, 16)] = _compact(keep, sk)
            need = _KOUT - e
            for c2 in range(7):
                sl = i16 + c2 * 16
                m = sl < need
                dst = base + e + c2 * 16
                oldi = oib[pl.ds(dst, 16)]
                oldv = ovb[pl.ds(dst, 16)]
                oib[pl.ds(dst, 16)] = jnp.where(m, sl, oldi)
                ovb[pl.ds(dst, 16)] = jnp.where(m, negv16, oldv)

        @pl.when(~fastok)
        def _slow_q():
            _slow_query(lq)
        return 0

    def _slow_query(lq):
        gq = lo + lq
        base = lq * _OPAD
        qlo = gq * _K
        qhi = qlo + _K

        def sel_cond(st):
            e, bkmin, bv, bk = st
            return (e < _KOUT) & (bv > jnp.float32(-1e30))

        def find_best(_ignored):
            def cmax(c, acc):
                ps = myp[pl.ds(c * 16, 16)]
                vs = vals[pl.ds(c * 16, 16)]
                m = (ps >= qlo) & (ps < qhi)
                return jnp.maximum(acc, jnp.where(m, vs, ninf16))
            bvv = lax.fori_loop(0, nch, cmax, ninf16)
            bv = lax.reduce_max_p.bind(bvv, axes=(0,))
            bvx = jnp.full((16,), bv, jnp.float32)

            def cmin(c, acc):
                ps = myp[pl.ds(c * 16, 16)]
                vs = vals[pl.ds(c * 16, 16)]
                m = (ps >= qlo) & (ps < qhi) & (vs == bvx)
                return jnp.minimum(acc, jnp.where(m, ps, 1 << 30))
            bpv = lax.fori_loop(0, nch, cmin, jnp.full((16,), 1 << 30, jnp.int32))
            bp = lax.reduce_min_p.bind(bpv, axes=(0,))
            return bv, bp

        def sel_body(st):
            e, bkmin, bv, bp = st
            bk = bp & 16383
            _sstore(oib, base + e, bk)
            _sstore(ovb, base + e, bv)
            bkmin = jnp.minimum(bkmin, bk)
            bpx = jnp.full((16,), bp, jnp.int32)

            def consume(c, _c):
                m = myp[pl.ds(c * 16, 16)] == bpx
                vc = vals[pl.ds(c * 16, 16)]
                vals[pl.ds(c * 16, 16)] = jnp.where(m, ninf16, vc)
                return 0
            lax.fori_loop(0, nch, consume, 0)
            nbv, nbp = find_best(0)
            return e + 1, bkmin, nbv, nbp

        bv0, bp0 = find_best(0)
        e, bkmin, _, _ = lax.while_loop(
            sel_cond, sel_body, (0, 1 << 30, bv0, bp0))

        @pl.when(e < _KOUT)
        def _fill():
            need = _KOUT - e

            @pl.when(bkmin >= 256)
            def _fast():
                def fout(c, _c):
                    sl = i16 + c * 16
                    m = sl < need
                    dst = base + e + c * 16
                    oldi = oib[pl.ds(dst, 16)]
                    oldv = ovb[pl.ds(dst, 16)]
                    oib[pl.ds(dst, 16)] = jnp.where(m, sl, oldi)
                    ovb[pl.ds(dst, 16)] = jnp.where(m, negv16, oldv)
                    return 0
                lax.fori_loop(0, (need + 15) // 16, fout, 0)

            @pl.when(bkmin < 256)
            def _slow():
                def zfl(c, _c):
                    fl[pl.ds(c * 16, 16)] = zero16
                    return 0
                lax.fori_loop(0, 16, zfl, 0)

                def setfl(j, _c):
                    kk = _sload(oib, base + j)

                    @pl.when(kk < 256)
                    def _s():
                        _sstore(fl, kk, 1)
                    return 0
                lax.fori_loop(0, e, setfl, 0)

                def fcomp(c, nf):
                    m = fl[pl.ds(c * 16, 16)] == 0
                    fb[pl.ds(nf, 16)] = _compact(m, i16 + c * 16)
                    return nf + _scount(m)
                lax.fori_loop(0, 16, fcomp, 0)

                def fout(c, _c):
                    sl = i16 + c * 16
                    m = sl < need
                    dst = base + e + c * 16
                    oldi = oib[pl.ds(dst, 16)]
                    oldv = ovb[pl.ds(dst, 16)]
                    oib[pl.ds(dst, 16)] = jnp.where(m, fb[pl.ds(c * 16, 16)],
                                                    oldi)
                    ovb[pl.ds(dst, 16)] = jnp.where(m, negv16, oldv)
                    return 0
                lax.fori_loop(0, (need + 15) // 16, fout, 0)
        return 0
    with jax.named_scope("b_pq"):
        lax.fori_loop(0, 32, per_query, 0)

    pltpu.async_copy(ovb.at[pl.ds(0, 32 * _OPAD)],
                     ov_ref.at[pl.ds(lo * _OPAD, 32 * _OPAD)], sem).wait()
    pltpu.async_copy(oib.at[pl.ds(0, 32 * _OPAD)],
                     oi_ref.at[pl.ds(lo * _OPAD, 32 * _OPAD)], sem).wait()


def kernel(queries, keys, W, b, max_candidates=100):
    del max_candidates
    wf = W.transpose(1, 0, 2).reshape(_D, _T * _H)
    bf = b.reshape(1, _T * _H)
    row = lax.broadcasted_iota(jnp.int32, (_T * _H, _T), 0)
    col = lax.broadcasted_iota(jnp.int32, (_T * _H, _T), 1)
    p2 = jnp.where(row // _H == col,
                   (2.0 ** (row % _H).astype(jnp.float32)), 0.0)
    x = jnp.concatenate([queries, keys], axis=0)
    n = _Q + _K

    codes = pl.pallas_call(
        _codes_body,
        grid=(n // 1024,),
        in_specs=[
            pl.BlockSpec((1024, _D), lambda i: (i, 0)),
            pl.BlockSpec((_D, _T * _H), lambda i: (0, 0)),
            pl.BlockSpec((1, _T * _H), lambda i: (0, 0)),
            pl.BlockSpec((_T * _H, _T), lambda i: (0, 0)),
        ],
        out_specs=pl.BlockSpec((1024, _T), lambda i: (i, 0)),
        out_shape=jax.ShapeDtypeStruct((n, _T), jnp.int32),
    )(x, wf, bf, p2)

    sims = pl.pallas_call(
        _sims_body,
        grid=(_K // 1024,),
        in_specs=[
            pl.BlockSpec((_Q, _D), lambda i: (0, 0)),
            pl.BlockSpec((1024, _D), lambda i: (i, 0)),
        ],
        out_specs=pl.BlockSpec((_Q, 8, 128), lambda i: (0, i, 0)),
        out_shape=jax.ShapeDtypeStruct((_Q, 128, 128), jnp.float32),
    )(queries, keys)

    code_q = codes[:_Q].T.reshape(_T * _Q)
    code_k = codes[_Q:].T.reshape(_T * _K)

    mesh = plsc.VectorSubcoreMesh(core_axis_name="c", subcore_axis_name="s")

    (pairs_p,) = pl.kernel(
        _match_body,
        mesh=mesh,
        compiler_params=pltpu.CompilerParams(needs_layout_passes=False),
        out_type=[
            jax.ShapeDtypeStruct((_NSUB * _ROW,), jnp.int32),
        ],
        scratch_types=[
            pltpu.VMEM((65536 + 16,), jnp.int32),
            pltpu.VMEM((_Q + 16,), jnp.int32),
            pltpu.VMEM((_Q + 16,), jnp.int32),
            pltpu.VMEM((4096 + 16,), jnp.int32),
            pltpu.VMEM((4096 + 16,), jnp.int32),
            pltpu.VMEM((_ROW,), jnp.int32),
            pltpu.SemaphoreType.DMA,
        ],
    )(code_q, code_k)

    simsv = sims.reshape(_Q * _K // 128, 128)

    ov, oi = pl.kernel(
        _select_body,
        mesh=mesh,
        compiler_params=pltpu.CompilerParams(needs_layout_passes=False),
        out_type=[
            jax.ShapeDtypeStruct((_Q * _OPAD,), jnp.float32),
            jax.ShapeDtypeStruct((_Q * _OPAD,), jnp.int32),
        ],
        scratch_types=[
            pltpu.VMEM((_NSUB * _ROW,), jnp.int32),
            pltpu.VMEM((_ROW,), jnp.int32),
            pltpu.VMEM((_ROW,), jnp.float32),
            pltpu.VMEM((_ROW,), jnp.int32),
            pltpu.VMEM((_CAP, 128), jnp.float32),
            pltpu.VMEM((512 + 16,), jnp.int32),
            pltpu.VMEM((512 + 16,), jnp.float32),
            pltpu.VMEM((48,), jnp.int32),
            pltpu.VMEM((48,), jnp.float32),
            pltpu.VMEM((48,), jnp.int32),
            pltpu.VMEM((256 + 16,), jnp.int32),
            pltpu.VMEM((256 + 16,), jnp.int32),
            pltpu.VMEM((32 * _OPAD + 16,), jnp.float32),
            pltpu.VMEM((32 * _OPAD + 16,), jnp.int32),
            pltpu.SemaphoreType.DMA,
        ],
    )(pairs_p, simsv)

    vals = ov.reshape(_Q, _OPAD)[:, :_KOUT]
    idx = oi.reshape(_Q, _OPAD)[:, :_KOUT]
    return vals, idx


# speedup vs baseline: 1.0794x; 1.0794x over previous
import functools

import jax
import jax.numpy as jnp
from jax import lax
from jax.experimental import pallas as pl
from jax.experimental.pallas import tpu as pltpu
from jax.experimental.pallas import tpu_sc as plsc

_Q, _K, _D = 1024, 16384, 256
_T, _H = 8, 16
_KOUT = 100
_OPAD = 112
_CAP = 256
_ROW = _CAP + 16
_NSUB = 32

_NEG = -1e9
_NINF = -3.0e38


def _codes_body(x_ref, wf_ref, bf_ref, p2_ref, code_ref):
    x = x_ref[...]
    proj = jnp.dot(x, wf_ref[...], preferred_element_type=jnp.float32)
    proj = proj + bf_ref[...]
    bits = (proj > 0).astype(jnp.float32)
    codef = jnp.dot(bits, p2_ref[...], preferred_element_type=jnp.float32)
    code_ref[...] = codef.astype(jnp.int32)


def _sims_body(q_ref, k_ref, o_ref):
    q = q_ref[...]
    for j in range(8):
        kb = k_ref[pl.ds(j * 128, 128), :]
        o_ref[:, j, :] = lax.dot_general(
            q, kb, (((1,), (1,)), ((), ())),
            preferred_element_type=jnp.float32)


def _iota16():
    return lax.iota(jnp.int32, 16)


def _scount(mask):
    return plsc.all_reduce_population_count(mask)[0]


def _sload(ref, i):
    return ref[pl.ds(i, 16)][0]


def _sstore(ref, i, val):
    old = ref[pl.ds(i, 16)]
    vv = jnp.full((16,), val)
    ref[pl.ds(i, 16)] = jnp.where(lax.iota(jnp.int32, 16) == 0, vv, old)


def _compact(m, payload):
    _, sv, _ = plsc.sort_key_val(lax.iota(jnp.int32, 16), payload, mask=m)
    return sv


def _match_body(cq_ref, ck_ref, pp_ref,
                qtab, nxt, qc, kc, hits, pp, sem):
    wid = lax.axis_index("s") * 2 + lax.axis_index("c")
    t = wid // 4
    quarter = wid % 4
    i16 = _iota16()
    neg16 = jnp.full((16,), -1, jnp.int32)

    pltpu.async_copy(cq_ref.at[pl.ds(t * _Q, _Q)],
                     qc.at[pl.ds(0, _Q)], sem).wait()
    pltpu.async_copy(ck_ref.at[pl.ds(t * _K + quarter * 4096, 4096)],
                     kc.at[pl.ds(0, 4096)], sem).wait()

    def zq(i, _):
        qtab[pl.ds(i * 16, 16)] = neg16
        return 0
    lax.fori_loop(0, 4096, zq, 0, unroll=8)

    def zpair(i, _):
        pp[pl.ds(i * 16, 16)] = neg16
        return 0
    lax.fori_loop(0, _ROW // 16, zpair, 0)

    def setbit(i, _):
        c = _sload(qc, i)
        _sstore(nxt, i, _sload(qtab, c))
        _sstore(qtab, c, i)
        return 0
    lax.fori_loop(0, _Q, setbit, 0)

    def scan(i, nh):
        k16 = kc[pl.ds(i * 16, 16)]
        wv = plsc.load_gather(qtab, [k16])
        hit = wv >= 0
        cnt = _scount(hit)

        @pl.when(cnt > 0)
        def _():
            hits[pl.ds(nh, 16)] = _compact(hit, i16 + i * 16)
        return nh + cnt
    nh = lax.fori_loop(0, 256, scan, 0)

    def per_hit(j, np_):
        hk = _sload(hits, j)
        ck = _sload(kc, hk)
        kg = hk + quarter * 4096

        def chain_cond(st):
            q, _n = st
            return q >= 0

        def chain_body(st):
            q, n = st
            _sstore(pp, jnp.minimum(n, _CAP), q * _K + kg)
            return _sload(nxt, q), n + 1
        _, np2 = lax.while_loop(chain_cond, chain_body,
                                (_sload(qtab, ck), np_))
        return np2
    lax.fori_loop(0, nh, per_hit, 0)

    pltpu.async_copy(pp, pp_ref.at[pl.ds(wid * _ROW, _ROW)], sem).wait()


def _select_body(pp_ref, sims_ref, ov_ref, oi_ref,
                 ap, myp, vals, rowidx, rows, buf2d, vbuf, cnt, shv, shp,
                 fl, fb, ovb, oib, sem):
    wid = lax.axis_index("s") * 2 + lax.axis_index("c")
    lo = wid * 32
    i16 = _iota16()
    zero16 = jnp.zeros((16,), jnp.int32)
    neg16 = jnp.full((16,), -1, jnp.int32)
    ninf16 = jnp.full((16,), _NINF, jnp.float32)
    negv16 = jnp.full((16,), _NEG, jnp.float32)

    with jax.named_scope("b_dma"):
        pltpu.async_copy(pp_ref, ap, sem).wait()

    def zinit(i, _):
        myp[pl.ds(i * 16, 16)] = neg16
        rowidx[pl.ds(i * 16, 16)] = i16 + i * 16
        return 0
    lax.fori_loop(0, _ROW // 16, zinit, 0)

    plo = lo * _K
    phi = (lo + 32) * _K

    def filt(c, np_):
        ps = ap[pl.ds(c * 16, 16)]
        m = (ps >= plo) & (ps < phi)
        myp[pl.ds(jnp.minimum(np_, _CAP), 16)] = _compact(m, ps)
        return np_ + _scount(m)
    with jax.named_scope("b_filt"):
        np_ = lax.fori_loop(0, _NSUB * _ROW // 16, filt, 0, unroll=4)
    tail = jnp.minimum(np_, _CAP)
    myp[pl.ds(tail, 16)] = neg16
    nch = (tail + 15) // 16

    def rowix(c, _):
        ps = myp[pl.ds(c * 16, 16)]
        r = jnp.where(ps >= 0, (ps >> 14) * 128 + ((ps & 16383) >> 7),
                      i16 + c * 16)
        rowidx[pl.ds(c * 16, 16)] = r
        return 0

    with jax.named_scope("b_rowix"):
        lax.fori_loop(0, nch, rowix, 0)
    for j in range(_CAP // 16):
        @pl.when(j * 16 < tail)
        def _issue(j=j):
            pltpu.async_copy(
                sims_ref.at[rowidx.at[pl.ds(j * 16, 16)]],
                rows.at[pl.ds(j * 16, 16)], sem)
    with jax.named_scope("b_gather"):
        for j in range(_CAP // 16):
            @pl.when(j * 16 < tail)
            def _drain(j=j):
                pltpu.make_async_copy(
                    sims_ref.at[rowidx.at[pl.ds(j * 16, 16)]],
                    rows.at[pl.ds(j * 16, 16)], sem).wait()

    def getval(c, _):
        pi = i16 + c * 16
        ps = myp[pl.ds(c * 16, 16)]
        v = plsc.load_gather(rows, [pi, ps & 127])
        vals[pl.ds(c * 16, 16)] = jnp.where(ps >= 0, v, ninf16)
        return 0
    lax.fori_loop(0, nch, getval, 0)

    def zcnt(i, _):
        cnt[pl.ds(i * 16, 16)] = zero16
        return 0
    lax.fori_loop(0, 3, zcnt, 0)

    def bpass(i, _):
        p = _sload(myp, i)
        lq = (p >> 14) - lo
        cv = _sload(cnt, lq)
        slot = lq * 16 + jnp.minimum(cv, 15)
        _sstore(buf2d, slot, p)
        _sstore(vbuf, slot, _sload(vals, i))
        _sstore(cnt, lq, cv + 1)
        return 0
    with jax.named_scope("b_bpass"):
        lax.fori_loop(0, jnp.minimum(np_, _CAP), bpass, 0)

    def per_query(lq, _):
        gq = lo + lq
        base = lq * _OPAD
        qlo = gq * _K
        qhi = qlo + _K

        c0 = _sload(cnt, lq)
        kv = buf2d[pl.ds(lq * 16, 16)]
        vv = vbuf[pl.ds(lq * 16, 16)]
        valid = i16 < c0
        sk, sp, _om = plsc.sort_key_val(vv, kv, mask=valid, descending=True)
        shv[pl.ds(0, 16)] = jnp.full((16,), 3.0e38, jnp.float32)
        shv[pl.ds(1, 16)] = sk
        shp[pl.ds(0, 16)] = neg16
        shp[pl.ds(1, 16)] = sp
        prevv = shv[pl.ds(0, 16)]
        prevp = shp[pl.ds(0, 16)]
        tie = valid & (sk == prevv) & (sp != prevp)
        dup = valid & (sp == prevp)
        keep = valid & (~dup)
        kk16 = sp & 16383
        anytie = _scount(tie) > 0
        anysmall = _scount(keep & (kk16 < 256)) > 0
        fastok = (c0 <= 16) & (~anytie) & (~anysmall)

        @pl.when(fastok)
        def _fast_q():
            e = _scount(keep)
            oib[pl.ds(base, 16)] = _compact(keep, kk16)
            ovb[pl.ds(base, 16)] = _compact(keep, sk)
            need = _KOUT - e
            for c2 in range(7):
                sl = i16 + c2 * 16
                m = sl < need
                dst = base + e + c2 * 16
                oldi = oib[pl.ds(dst, 16)]
                oldv = ovb[pl.ds(dst, 16)]
                oib[pl.ds(dst, 16)] = jnp.where(m, sl, oldi)
                ovb[pl.ds(dst, 16)] = jnp.where(m, negv16, oldv)

        @pl.when(~fastok)
        def _slow_q():
            _slow_query(lq)
        return 0

    def _slow_query(lq):
        gq = lo + lq
        base = lq * _OPAD
        qlo = gq * _K
        qhi = qlo + _K

        def sel_cond(st):
            e, bkmin, bv, bk = st
            return (e < _KOUT) & (bv > jnp.float32(-1e30))

        def find_best(_ignored):
            def cmax(c, acc):
                ps = myp[pl.ds(c * 16, 16)]
                vs = vals[pl.ds(c * 16, 16)]
                m = (ps >= qlo) & (ps < qhi)
                return jnp.maximum(acc, jnp.where(m, vs, ninf16))
            bvv = lax.fori_loop(0, nch, cmax, ninf16)
            bv = lax.reduce_max_p.bind(bvv, axes=(0,))
            bvx = jnp.full((16,), bv, jnp.float32)

            def cmin(c, acc):
                ps = myp[pl.ds(c * 16, 16)]
                vs = vals[pl.ds(c * 16, 16)]
                m = (ps >= qlo) & (ps < qhi) & (vs == bvx)
                return jnp.minimum(acc, jnp.where(m, ps, 1 << 30))
            bpv = lax.fori_loop(0, nch, cmin, jnp.full((16,), 1 << 30, jnp.int32))
            bp = lax.reduce_min_p.bind(bpv, axes=(0,))
            return bv, bp

        def sel_body(st):
            e, bkmin, bv, bp = st
            bk = bp & 16383
            _sstore(oib, base + e, bk)
            _sstore(ovb, base + e, bv)
            bkmin = jnp.minimum(bkmin, bk)
            bpx = jnp.full((16,), bp, jnp.int32)

            def consume(c, _c):
                m = myp[pl.ds(c * 16, 16)] == bpx
                vc = vals[pl.ds(c * 16, 16)]
                vals[pl.ds(c * 16, 16)] = jnp.where(m, ninf16, vc)
                return 0
            lax.fori_loop(0, nch, consume, 0)
            nbv, nbp = find_best(0)
            return e + 1, bkmin, nbv, nbp

        bv0, bp0 = find_best(0)
        e, bkmin, _, _ = lax.while_loop(
            sel_cond, sel_body, (0, 1 << 30, bv0, bp0))

        @pl.when(e < _KOUT)
        def _fill():
            need = _KOUT - e

            @pl.when(bkmin >= 256)
            def _fast():
                def fout(c, _c):
                    sl = i16 + c * 16
                    m = sl < need
                    dst = base + e + c * 16
                    oldi = oib[pl.ds(dst, 16)]
                    oldv = ovb[pl.ds(dst, 16)]
                    oib[pl.ds(dst, 16)] = jnp.where(m, sl, oldi)
                    ovb[pl.ds(dst, 16)] = jnp.where(m, negv16, oldv)
                    return 0
                lax.fori_loop(0, (need + 15) // 16, fout, 0)

            @pl.when(bkmin < 256)
            def _slow():
                def zfl(c, _c):
                    fl[pl.ds(c * 16, 16)] = zero16
                    return 0
                lax.fori_loop(0, 16, zfl, 0)

                def setfl(j, _c):
                    kk = _sload(oib, base + j)

                    @pl.when(kk < 256)
                    def _s():
                        _sstore(fl, kk, 1)
                    return 0
                lax.fori_loop(0, e, setfl, 0)

                def fcomp(c, nf):
                    m = fl[pl.ds(c * 16, 16)] == 0
                    fb[pl.ds(nf, 16)] = _compact(m, i16 + c * 16)
                    return nf + _scount(m)
                lax.fori_loop(0, 16, fcomp, 0)

                def fout(c, _c):
                    sl = i16 + c * 16
                    m = sl < need
                    dst = base + e + c * 16
                    oldi = oib[pl.ds(dst, 16)]
                    oldv = ovb[pl.ds(dst, 16)]
                    oib[pl.ds(dst, 16)] = jnp.where(m, fb[pl.ds(c * 16, 16)],
                                                    oldi)
                    ovb[pl.ds(dst, 16)] = jnp.where(m, negv16, oldv)
                    return 0
                lax.fori_loop(0, (need + 15) // 16, fout, 0)
        return 0
    with jax.named_scope("b_pq"):
        lax.fori_loop(0, 32, per_query, 0)

    pltpu.async_copy(ovb.at[pl.ds(0, 32 * _OPAD)],
                     ov_ref.at[pl.ds(lo * _OPAD, 32 * _OPAD)], sem).wait()
    pltpu.async_copy(oib.at[pl.ds(0, 32 * _OPAD)],
                     oi_ref.at[pl.ds(lo * _OPAD, 32 * _OPAD)], sem).wait()


def kernel(queries, keys, W, b, max_candidates=100):
    del max_candidates
    wf = W.transpose(1, 0, 2).reshape(_D, _T * _H)
    bf = b.reshape(1, _T * _H)
    row = lax.broadcasted_iota(jnp.int32, (_T * _H, _T), 0)
    col = lax.broadcasted_iota(jnp.int32, (_T * _H, _T), 1)
    p2 = jnp.where(row // _H == col,
                   (2.0 ** (row % _H).astype(jnp.float32)), 0.0)
    x = jnp.concatenate([queries, keys], axis=0)
    n = _Q + _K

    codes = pl.pallas_call(
        _codes_body,
        grid=(n // 1024,),
        in_specs=[
            pl.BlockSpec((1024, _D), lambda i: (i, 0)),
            pl.BlockSpec((_D, _T * _H), lambda i: (0, 0)),
            pl.BlockSpec((1, _T * _H), lambda i: (0, 0)),
            pl.BlockSpec((_T * _H, _T), lambda i: (0, 0)),
        ],
        out_specs=pl.BlockSpec((1024, _T), lambda i: (i, 0)),
        out_shape=jax.ShapeDtypeStruct((n, _T), jnp.int32),
    )(x, wf, bf, p2)

    sims = pl.pallas_call(
        _sims_body,
        grid=(_K // 1024,),
        in_specs=[
            pl.BlockSpec((_Q, _D), lambda i: (0, 0)),
            pl.BlockSpec((1024, _D), lambda i: (i, 0)),
        ],
        out_specs=pl.BlockSpec((_Q, 8, 128), lambda i: (0, i, 0)),
        out_shape=jax.ShapeDtypeStruct((_Q, 128, 128), jnp.float32),
    )(queries, keys)

    code_q = codes[:_Q].T.reshape(_T * _Q)
    code_k = codes[_Q:].T.reshape(_T * _K)

    mesh = plsc.VectorSubcoreMesh(core_axis_name="c", subcore_axis_name="s")

    (pairs_p,) = pl.kernel(
        _match_body,
        mesh=mesh,
        compiler_params=pltpu.CompilerParams(needs_layout_passes=False),
        out_type=[
            jax.ShapeDtypeStruct((_NSUB * _ROW,), jnp.int32),
        ],
        scratch_types=[
            pltpu.VMEM((65536 + 16,), jnp.int32),
            pltpu.VMEM((_Q + 16,), jnp.int32),
            pltpu.VMEM((_Q + 16,), jnp.int32),
            pltpu.VMEM((4096 + 16,), jnp.int32),
            pltpu.VMEM((4096 + 16,), jnp.int32),
            pltpu.VMEM((_ROW,), jnp.int32),
            pltpu.SemaphoreType.DMA,
        ],
    )(code_q, code_k)

    simsv = sims.reshape(_Q * _K // 128, 128)

    ov, oi = pl.kernel(
        _select_body,
        mesh=mesh,
        compiler_params=pltpu.CompilerParams(needs_layout_passes=False),
        out_type=[
            jax.ShapeDtypeStruct((_Q * _OPAD,), jnp.float32),
            jax.ShapeDtypeStruct((_Q * _OPAD,), jnp.int32),
        ],
        scratch_types=[
            pltpu.VMEM((_NSUB * _ROW,), jnp.int32),
            pltpu.VMEM((_ROW,), jnp.int32),
            pltpu.VMEM((_ROW,), jnp.float32),
            pltpu.VMEM((_ROW,), jnp.int32),
            pltpu.VMEM((_CAP, 128), jnp.float32),
            pltpu.VMEM((512 + 16,), jnp.int32),
            pltpu.VMEM((512 + 16,), jnp.float32),
            pltpu.VMEM((48,), jnp.int32),
            pltpu.VMEM((48,), jnp.float32),
            pltpu.VMEM((48,), jnp.int32),
            pltpu.VMEM((256 + 16,), jnp.int32),
            pltpu.VMEM((256 + 16,), jnp.int32),
            pltpu.VMEM((32 * _OPAD + 16,), jnp.float32),
            pltpu.VMEM((32 * _OPAD + 16,), jnp.int32),
            pltpu.SemaphoreType.DMA,
        ],
    )(pairs_p, simsv)

    vals = ov.reshape(_Q, _OPAD)[:, :_KOUT]
    idx = oi.reshape(_Q, _OPAD)[:, :_KOUT]
    return vals, idx
